# Initial kernel scaffold; baseline (speedup 1.0000x reference)
import jax
import jax.numpy as jnp
from jax.experimental import pallas as pl


def kernel(idx_x, idx_w, x_batch, edge_index, edge_weight, word_vec, W_rel, b_rel, W_root, bn1_g, bn1_b, fc1_W, fc1_b, fc2_W, fc2_b, mean_W, mean_b, mean_bn_b, logvar_W, logvar_b, phi_W, phi_b):
    raise NotImplementedError("write your pallas kernel here")


# speedup vs baseline: 8.8849x; 8.8849x over previous
import functools

import jax
import jax.numpy as jnp
from jax import lax
from jax.experimental import pallas as pl
from jax.experimental.pallas import tpu as pltpu
from jax.experimental.pallas import tpu_sc as plsc

N = 10000
E = 320000
VOCAB = 128
NW = 64
DCAT = VOCAB + NW
ENC_NH = 192
NT = 192
NB = 64
EPS = 1e-5

_NC = 2
_NS = 16
_ROWS = N // _NS
_EPC = E // _NC
_CH = 8000
_NCHUNK = _EPC // _CH
_ACC = _ROWS * VOCAB

@functools.cache
def _agg_sc_call():
    mesh = plsc.VectorSubcoreMesh(core_axis_name="c", subcore_axis_name="s",
                                  num_cores=_NC, num_subcores=_NS)
    return functools.partial(
        pl.kernel,
        out_type=jax.ShapeDtypeStruct((_NC, _NS, _ACC), jnp.float32),
        mesh=mesh,
        compiler_params=pltpu.CompilerParams(needs_layout_passes=False),
        scratch_types=[
            pltpu.VMEM((N,), jnp.int32),
            pltpu.VMEM((_CH,), jnp.int32),
            pltpu.VMEM((_CH,), jnp.int32),
            pltpu.VMEM((_CH,), jnp.float32),
            pltpu.VMEM((_ACC,), jnp.float32),
        ],
    )(_agg_sc_body)


def _agg_sc_body(src_hbm, dst_hbm, ew_hbm, idxx_hbm, out_hbm,
                 idxx_v, src_v, dst_v, ew_v, acc_v):
    c = lax.axis_index("c")
    s = lax.axis_index("s")
    row_base = s * _ROWS

    def zero_body(i, _):
        acc_v[pl.ds(i * 16, 16)] = jnp.zeros((16,), jnp.float32)
        return 0
    lax.fori_loop(0, _ACC // 16, zero_body, 0)

    pltpu.sync_copy(idxx_hbm, idxx_v)

    def chunk_body(k, _):
        off = c * _EPC + k * _CH
        pltpu.sync_copy(src_hbm.at[pl.ds(off, _CH)], src_v)
        pltpu.sync_copy(dst_hbm.at[pl.ds(off, _CH)], dst_v)
        pltpu.sync_copy(ew_hbm.at[pl.ds(off, _CH)], ew_v)

        def edge_body(i, _):
            s16 = src_v[pl.ds(i * 16, 16)]
            d16 = dst_v[pl.ds(i * 16, 16)]
            w16 = ew_v[pl.ds(i * 16, 16)]
            c16 = plsc.load_gather(idxx_v, [s16])
            rel = d16 - row_base
            m = (rel >= 0) & (rel < _ROWS)
            flat = jnp.where(m, rel * VOCAB + c16, 0)
            plsc.addupdate_scatter(acc_v, [flat], w16, mask=m)
            return 0
        lax.fori_loop(0, _CH // 16, edge_body, 0)
        return 0
    lax.fori_loop(0, _NCHUNK, chunk_body, 0)

    pltpu.sync_copy(acc_v, out_hbm.at[c, s])


_BR = 1000
_GRID = N // _BR


def _onehot(ids, width):
    return (ids[:, None] == lax.broadcasted_iota(jnp.int32, (ids.shape[0], width), 1)
            ).astype(jnp.float32)


def _conv_body(parts_ref, idx_ref, wsh_ref, wrel_ref, wroot_ref,
               conv_ref, sum_ref, ssq_ref):
    ids = idx_ref[0, 0, :]
    wsh = wsh_ref[0, 0, :]
    oh = _onehot(ids, VOCAB)
    agg = parts_ref[0] + parts_ref[1] + oh * wsh[:, None]
    conv = (jnp.dot(agg, wrel_ref[...], preferred_element_type=jnp.float32)
            + jnp.dot(oh, wroot_ref[...], preferred_element_type=jnp.float32))
    conv_ref[...] = conv

    @pl.when(pl.program_id(0) == 0)
    def _():
        sum_ref[...] = jnp.zeros_like(sum_ref)
        ssq_ref[...] = jnp.zeros_like(ssq_ref)

    sum_ref[...] += jnp.broadcast_to(jnp.sum(conv, axis=0)[None, :], (8, NW))
    ssq_ref[...] += jnp.broadcast_to(jnp.sum(conv * conv, axis=0)[None, :], (8, NW))


def _enc_body(conv_ref, idx_ref, xb_ref, scale_ref, shift_ref,
              fc1w_ref, fc2w_ref, fc1b_ref, fc2b_ref,
              enc1_ref, g_ref):
    enc1 = jnp.tanh(conv_ref[...] * scale_ref[0:1, :] + shift_ref[0:1, :])
    oh = _onehot(idx_ref[0, 0, :], VOCAB)
    z1 = (jnp.dot(enc1, fc1w_ref[:NW], preferred_element_type=jnp.float32)
          + jnp.dot(oh, fc1w_ref[NW:], preferred_element_type=jnp.float32)
          + fc1b_ref[0:1, :])
    z2 = (jnp.dot(enc1, fc2w_ref[:NW], preferred_element_type=jnp.float32)
          + jnp.dot(oh, fc2w_ref[NW:], preferred_element_type=jnp.float32)
          + fc2b_ref[0:1, :])
    enc2 = jax.nn.sigmoid(z1) * jnp.tanh(z2)
    ohb = _onehot(xb_ref[0, 0, :], NB)
    enc1_ref[...] = enc1

    @pl.when(pl.program_id(0) == 0)
    def _():
        g_ref[...] = jnp.zeros_like(g_ref)

    g_ref[...] += lax.dot_general(ohb, enc2, (((0,), (0,)), ((), ())),
                                  preferred_element_type=jnp.float32)


def _head_body(g_ref, meanw_ref, meanb_ref, mbnb_ref, lvw_ref, lvb_ref, phiw_ref,
               mean_ref, logvar_ref, gp_ref):
    g = g_ref[...]
    m = jnp.dot(g, meanw_ref[...], preferred_element_type=jnp.float32) + meanb_ref[0:1, :]
    mu = jnp.mean(m, axis=0, keepdims=True)
    var = jnp.mean(m * m, axis=0, keepdims=True) - mu * mu
    mean_ref[...] = (m - mu) * lax.rsqrt(var + EPS) + mbnb_ref[0:1, :]
    logvar_ref[...] = jnp.dot(g, lvw_ref[...], preferred_element_type=jnp.float32) + lvb_ref[0:1, :]
    gp_ref[...] = jnp.dot(g, phiw_ref[DCAT:], preferred_element_type=jnp.float32)


def _phi_body(enc1_ref, idx_ref, xb_ref, phiw_ref, gp_ref, phib_ref, phi_ref):
    oh = _onehot(idx_ref[0, 0, :], VOCAB)
    ohb = _onehot(xb_ref[0, 0, :], NB)
    logits = (jnp.dot(enc1_ref[...], phiw_ref[:NW], preferred_element_type=jnp.float32)
              + jnp.dot(oh, phiw_ref[NW:DCAT], preferred_element_type=jnp.float32)
              + jnp.dot(ohb, gp_ref[...], preferred_element_type=jnp.float32)
              + phib_ref[0:1, :])
    mx = jnp.max(logits, axis=-1, keepdims=True)
    e = jnp.exp(logits - mx)
    phi_ref[...] = e / jnp.sum(e, axis=-1, keepdims=True)


def _row8(v, k):
    return jnp.broadcast_to(v[None, :], (8, k))


def kernel(idx_x, idx_w, x_batch, edge_index, edge_weight, word_vec,
           W_rel, b_rel, W_root, bn1_g, bn1_b, fc1_W, fc1_b, fc2_W, fc2_b,
           mean_W, mean_b, mean_bn_b, logvar_W, logvar_b, phi_W, phi_b):
    idx_x = idx_x.astype(jnp.int32)
    x_batch = x_batch.astype(jnp.int32)
    src = edge_index[0].astype(jnp.int32)
    dst = edge_index[1].astype(jnp.int32)
    ew = edge_weight.astype(jnp.float32)

    parts = _agg_sc_call()(src, dst, ew, idx_x).reshape(_NC, N, VOCAB)

    idx3 = idx_x.reshape(_GRID, 1, _BR)
    xb3 = x_batch.reshape(_GRID, 1, _BR)
    wsh3 = jnp.concatenate([jnp.zeros((1,), jnp.float32), idx_w[:-1]]).reshape(_GRID, 1, _BR)

    conv, sums, ssq = pl.pallas_call(
        _conv_body,
        grid=(_GRID,),
        in_specs=[
            pl.BlockSpec((_NC, _BR, VOCAB), lambda i: (0, i, 0)),
            pl.BlockSpec((1, 1, _BR), lambda i: (i, 0, 0)),
            pl.BlockSpec((1, 1, _BR), lambda i: (i, 0, 0)),
            pl.BlockSpec((VOCAB, NW), lambda i: (0, 0)),
            pl.BlockSpec((VOCAB, NW), lambda i: (0, 0)),
        ],
        out_specs=[
            pl.BlockSpec((_BR, NW), lambda i: (i, 0)),
            pl.BlockSpec((8, NW), lambda i: (0, 0)),
            pl.BlockSpec((8, NW), lambda i: (0, 0)),
        ],
        out_shape=[
            jax.ShapeDtypeStruct((N, NW), jnp.float32),
            jax.ShapeDtypeStruct((8, NW), jnp.float32),
            jax.ShapeDtypeStruct((8, NW), jnp.float32),
        ],
    )(parts, idx3, wsh3, W_rel, W_root)

    mu = sums[0] / N
    var = ssq[0] / N - mu * mu
    scale = bn1_g * lax.rsqrt(var + EPS)
    shift = bn1_b - mu * scale

    enc1, enc2g = pl.pallas_call(
        _enc_body,
        grid=(_GRID,),
        in_specs=[
            pl.BlockSpec((_BR, NW), lambda i: (i, 0)),
            pl.BlockSpec((1, 1, _BR), lambda i: (i, 0, 0)),
            pl.BlockSpec((1, 1, _BR), lambda i: (i, 0, 0)),
            pl.BlockSpec((8, NW), lambda i: (0, 0)),
            pl.BlockSpec((8, NW), lambda i: (0, 0)),
            pl.BlockSpec((DCAT, ENC_NH), lambda i: (0, 0)),
            pl.BlockSpec((ENC_NH, DCAT), lambda i: (0, 0)),
            pl.BlockSpec((8, ENC_NH), lambda i: (0, 0)),
            pl.BlockSpec((8, DCAT), lambda i: (0, 0)),
        ],
        out_specs=[
            pl.BlockSpec((_BR, NW), lambda i: (i, 0)),
            pl.BlockSpec((NB, DCAT), lambda i: (0, 0)),
        ],
        out_shape=[
            jax.ShapeDtypeStruct((N, NW), jnp.float32),
            jax.ShapeDtypeStruct((NB, DCAT), jnp.float32),
        ],
    )(conv, idx3, xb3, _row8(scale, NW), _row8(shift, NW),
      fc1_W, fc2_W, _row8(fc1_b, ENC_NH), _row8(fc2_b, DCAT))

    mean, logvar, gphi = pl.pallas_call(
        _head_body,
        out_shape=[
            jax.ShapeDtypeStruct((NB, NT), jnp.float32),
            jax.ShapeDtypeStruct((NB, ENC_NH), jnp.float32),
            jax.ShapeDtypeStruct((NB, NT), jnp.float32),
        ],
    )(enc2g, mean_W, _row8(mean_b, NT), _row8(mean_bn_b, NT),
      logvar_W, _row8(logvar_b, ENC_NH), phi_W)

    phi = pl.pallas_call(
        _phi_body,
        grid=(_GRID,),
        in_specs=[
            pl.BlockSpec((_BR, NW), lambda i: (i, 0)),
            pl.BlockSpec((1, 1, _BR), lambda i: (i, 0, 0)),
            pl.BlockSpec((1, 1, _BR), lambda i: (i, 0, 0)),
            pl.BlockSpec((DCAT + NT, NT), lambda i: (0, 0)),
            pl.BlockSpec((NB, NT), lambda i: (0, 0)),
            pl.BlockSpec((8, NT), lambda i: (0, 0)),
        ],
        out_specs=pl.BlockSpec((_BR, NT), lambda i: (i, 0)),
        out_shape=jax.ShapeDtypeStruct((N, NT), jnp.float32),
    )(enc1, idx3, xb3, phi_W, gphi, _row8(phi_b, NT))

    return (mean, logvar, phi)


# speedup vs baseline: 14.4401x; 1.6252x over previous
import functools

import jax
import jax.numpy as jnp
from jax import lax
from jax.experimental import pallas as pl
from jax.experimental.pallas import tpu as pltpu
from jax.experimental.pallas import tpu_sc as plsc

N = 10000
E = 320000
VOCAB = 128
NW = 64
DCAT = VOCAB + NW
ENC_NH = 192
NT = 192
NB = 64
EPS = 1e-5

_NC = 2
_NS = 16
_ROWS = N // _NS
_EPC = E // _NC
_CH = 8000
_NCHUNK = _EPC // _CH
_ACC = _ROWS * VOCAB

@functools.cache
def _agg_sc_call():
    mesh = plsc.VectorSubcoreMesh(core_axis_name="c", subcore_axis_name="s",
                                  num_cores=_NC, num_subcores=_NS)
    return functools.partial(
        pl.kernel,
        out_type=jax.ShapeDtypeStruct((_NC, _NS, _ACC), jnp.float32),
        mesh=mesh,
        compiler_params=pltpu.CompilerParams(needs_layout_passes=False),
        scratch_types=[
            pltpu.VMEM((N,), jnp.int32),
            pltpu.VMEM((_CH,), jnp.int32),
            pltpu.VMEM((_CH,), jnp.int32),
            pltpu.VMEM((_CH,), jnp.float32),
            pltpu.VMEM((_ACC,), jnp.float32),
        ],
    )(_agg_sc_body)


def _agg_sc_body(src_hbm, dst_hbm, ew_hbm, idxx_hbm, out_hbm,
                 idxx_v, src_v, dst_v, ew_v, acc_v):
    c = lax.axis_index("c")
    s = lax.axis_index("s")
    row_base = s * _ROWS

    @plsc.parallel_loop(0, _ACC, 16, unroll=8)
    def _zero(i):
        acc_v[pl.ds(i, 16)] = jnp.zeros((16,), jnp.float32)

    pltpu.sync_copy(idxx_hbm, idxx_v)

    def chunk_body(k, _):
        off = c * _EPC + k * _CH
        pltpu.sync_copy(src_hbm.at[pl.ds(off, _CH)], src_v)
        pltpu.sync_copy(dst_hbm.at[pl.ds(off, _CH)], dst_v)
        pltpu.sync_copy(ew_hbm.at[pl.ds(off, _CH)], ew_v)

        @plsc.parallel_loop(0, _CH, 16, unroll=8)
        def _edges(i):
            s16 = src_v[pl.ds(i, 16)]
            d16 = dst_v[pl.ds(i, 16)]
            w16 = ew_v[pl.ds(i, 16)]
            c16 = plsc.load_gather(idxx_v, [s16])
            rel = d16 - row_base
            m = (rel >= 0) & (rel < _ROWS)
            flat = jnp.where(m, rel * VOCAB + c16, 0)
            plsc.addupdate_scatter(acc_v, [flat], w16, mask=m)
        return 0
    lax.fori_loop(0, _NCHUNK, chunk_body, 0)

    pltpu.sync_copy(acc_v, out_hbm.at[c, s])


_BR = 1000
_GRID = N // _BR


def _onehot(ids, width):
    return (ids[:, None] == lax.broadcasted_iota(jnp.int32, (ids.shape[0], width), 1)
            ).astype(jnp.float32)


def _conv_body(parts_ref, idx_ref, wsh_ref, wrel_ref, wroot_ref,
               conv_ref, sum_ref, ssq_ref):
    ids = idx_ref[0, 0, :]
    wsh = wsh_ref[0, 0, :]
    oh = _onehot(ids, VOCAB)
    agg = parts_ref[0] + parts_ref[1] + oh * wsh[:, None]
    conv = (jnp.dot(agg, wrel_ref[...], preferred_element_type=jnp.float32)
            + jnp.dot(oh, wroot_ref[...], preferred_element_type=jnp.float32))
    conv_ref[...] = conv

    @pl.when(pl.program_id(0) == 0)
    def _():
        sum_ref[...] = jnp.zeros_like(sum_ref)
        ssq_ref[...] = jnp.zeros_like(ssq_ref)

    sum_ref[...] += jnp.broadcast_to(jnp.sum(conv, axis=0)[None, :], (8, NW))
    ssq_ref[...] += jnp.broadcast_to(jnp.sum(conv * conv, axis=0)[None, :], (8, NW))


def _enc_body(conv_ref, idx_ref, xb_ref, scale_ref, shift_ref,
              fc1w_ref, fc2w_ref, fc1b_ref, fc2b_ref,
              enc1_ref, g_ref):
    enc1 = jnp.tanh(conv_ref[...] * scale_ref[0:1, :] + shift_ref[0:1, :])
    oh = _onehot(idx_ref[0, 0, :], VOCAB)
    z1 = (jnp.dot(enc1, fc1w_ref[:NW], preferred_element_type=jnp.float32)
          + jnp.dot(oh, fc1w_ref[NW:], preferred_element_type=jnp.float32)
          + fc1b_ref[0:1, :])
    z2 = (jnp.dot(enc1, fc2w_ref[:NW], preferred_element_type=jnp.float32)
          + jnp.dot(oh, fc2w_ref[NW:], preferred_element_type=jnp.float32)
          + fc2b_ref[0:1, :])
    enc2 = jax.nn.sigmoid(z1) * jnp.tanh(z2)
    ohb = _onehot(xb_ref[0, 0, :], NB)
    enc1_ref[...] = enc1

    @pl.when(pl.program_id(0) == 0)
    def _():
        g_ref[...] = jnp.zeros_like(g_ref)

    g_ref[...] += lax.dot_general(ohb, enc2, (((0,), (0,)), ((), ())),
                                  preferred_element_type=jnp.float32)


def _head_body(g_ref, meanw_ref, meanb_ref, mbnb_ref, lvw_ref, lvb_ref, phiw_ref,
               mean_ref, logvar_ref, gp_ref):
    g = g_ref[...]
    m = jnp.dot(g, meanw_ref[...], preferred_element_type=jnp.float32) + meanb_ref[0:1, :]
    mu = jnp.mean(m, axis=0, keepdims=True)
    var = jnp.mean(m * m, axis=0, keepdims=True) - mu * mu
    mean_ref[...] = (m - mu) * lax.rsqrt(var + EPS) + mbnb_ref[0:1, :]
    logvar_ref[...] = jnp.dot(g, lvw_ref[...], preferred_element_type=jnp.float32) + lvb_ref[0:1, :]
    gp_ref[...] = jnp.dot(g, phiw_ref[DCAT:], preferred_element_type=jnp.float32)


def _phi_body(enc1_ref, idx_ref, xb_ref, phiw_ref, gp_ref, phib_ref, phi_ref):
    oh = _onehot(idx_ref[0, 0, :], VOCAB)
    ohb = _onehot(xb_ref[0, 0, :], NB)
    logits = (jnp.dot(enc1_ref[...], phiw_ref[:NW], preferred_element_type=jnp.float32)
              + jnp.dot(oh, phiw_ref[NW:DCAT], preferred_element_type=jnp.float32)
              + jnp.dot(ohb, gp_ref[...], preferred_element_type=jnp.float32)
              + phib_ref[0:1, :])
    mx = jnp.max(logits, axis=-1, keepdims=True)
    e = jnp.exp(logits - mx)
    phi_ref[...] = e / jnp.sum(e, axis=-1, keepdims=True)


def _row8(v, k):
    return jnp.broadcast_to(v[None, :], (8, k))


def kernel(idx_x, idx_w, x_batch, edge_index, edge_weight, word_vec,
           W_rel, b_rel, W_root, bn1_g, bn1_b, fc1_W, fc1_b, fc2_W, fc2_b,
           mean_W, mean_b, mean_bn_b, logvar_W, logvar_b, phi_W, phi_b):
    idx_x = idx_x.astype(jnp.int32)
    x_batch = x_batch.astype(jnp.int32)
    src = edge_index[0].astype(jnp.int32)
    dst = edge_index[1].astype(jnp.int32)
    ew = edge_weight.astype(jnp.float32)

    parts = _agg_sc_call()(src, dst, ew, idx_x).reshape(_NC, N, VOCAB)

    idx3 = idx_x.reshape(_GRID, 1, _BR)
    xb3 = x_batch.reshape(_GRID, 1, _BR)
    wsh3 = jnp.concatenate([jnp.zeros((1,), jnp.float32), idx_w[:-1]]).reshape(_GRID, 1, _BR)

    conv, sums, ssq = pl.pallas_call(
        _conv_body,
        grid=(_GRID,),
        in_specs=[
            pl.BlockSpec((_NC, _BR, VOCAB), lambda i: (0, i, 0)),
            pl.BlockSpec((1, 1, _BR), lambda i: (i, 0, 0)),
            pl.BlockSpec((1, 1, _BR), lambda i: (i, 0, 0)),
            pl.BlockSpec((VOCAB, NW), lambda i: (0, 0)),
            pl.BlockSpec((VOCAB, NW), lambda i: (0, 0)),
        ],
        out_specs=[
            pl.BlockSpec((_BR, NW), lambda i: (i, 0)),
            pl.BlockSpec((8, NW), lambda i: (0, 0)),
            pl.BlockSpec((8, NW), lambda i: (0, 0)),
        ],
        out_shape=[
            jax.ShapeDtypeStruct((N, NW), jnp.float32),
            jax.ShapeDtypeStruct((8, NW), jnp.float32),
            jax.ShapeDtypeStruct((8, NW), jnp.float32),
        ],
    )(parts, idx3, wsh3, W_rel, W_root)

    mu = sums[0] / N
    var = ssq[0] / N - mu * mu
    scale = bn1_g * lax.rsqrt(var + EPS)
    shift = bn1_b - mu * scale

    enc1, enc2g = pl.pallas_call(
        _enc_body,
        grid=(_GRID,),
        in_specs=[
            pl.BlockSpec((_BR, NW), lambda i: (i, 0)),
            pl.BlockSpec((1, 1, _BR), lambda i: (i, 0, 0)),
            pl.BlockSpec((1, 1, _BR), lambda i: (i, 0, 0)),
            pl.BlockSpec((8, NW), lambda i: (0, 0)),
            pl.BlockSpec((8, NW), lambda i: (0, 0)),
            pl.BlockSpec((DCAT, ENC_NH), lambda i: (0, 0)),
            pl.BlockSpec((ENC_NH, DCAT), lambda i: (0, 0)),
            pl.BlockSpec((8, ENC_NH), lambda i: (0, 0)),
            pl.BlockSpec((8, DCAT), lambda i: (0, 0)),
        ],
        out_specs=[
            pl.BlockSpec((_BR, NW), lambda i: (i, 0)),
            pl.BlockSpec((NB, DCAT), lambda i: (0, 0)),
        ],
        out_shape=[
            jax.ShapeDtypeStruct((N, NW), jnp.float32),
            jax.ShapeDtypeStruct((NB, DCAT), jnp.float32),
        ],
    )(conv, idx3, xb3, _row8(scale, NW), _row8(shift, NW),
      fc1_W, fc2_W, _row8(fc1_b, ENC_NH), _row8(fc2_b, DCAT))

    mean, logvar, gphi = pl.pallas_call(
        _head_body,
        out_shape=[
            jax.ShapeDtypeStruct((NB, NT), jnp.float32),
            jax.ShapeDtypeStruct((NB, ENC_NH), jnp.float32),
            jax.ShapeDtypeStruct((NB, NT), jnp.float32),
        ],
    )(enc2g, mean_W, _row8(mean_b, NT), _row8(mean_bn_b, NT),
      logvar_W, _row8(logvar_b, ENC_NH), phi_W)

    phi = pl.pallas_call(
        _phi_body,
        grid=(_GRID,),
        in_specs=[
            pl.BlockSpec((_BR, NW), lambda i: (i, 0)),
            pl.BlockSpec((1, 1, _BR), lambda i: (i, 0, 0)),
            pl.BlockSpec((1, 1, _BR), lambda i: (i, 0, 0)),
            pl.BlockSpec((DCAT + NT, NT), lambda i: (0, 0)),
            pl.BlockSpec((NB, NT), lambda i: (0, 0)),
            pl.BlockSpec((8, NT), lambda i: (0, 0)),
        ],
        out_specs=pl.BlockSpec((_BR, NT), lambda i: (i, 0)),
        out_shape=jax.ShapeDtypeStruct((N, NT), jnp.float32),
    )(enc1, idx3, xb3, phi_W, gphi, _row8(phi_b, NT))

    return (mean, logvar, phi)
